# baseline (speedup 1.0000x reference)
import jax
import jax.numpy as jnp
from jax import lax
from jax.experimental import pallas as pl
from jax.experimental.pallas import tpu as pltpu
from jax.experimental.pallas import tpu_sc as plsc

VOCAB = 1000000
EMBED_DIM = 32
BATCH = 4096
HIST = 200

NC = 2
NS = 16
NW = NC * NS
BPW = BATCH // NW
DT = EMBED_DIM // 8
NBUF = 2
L = 16


def _emb_body(xt_hbm, table_hbm, out_hbm, idxt_v, rows_v, tr_v, gsems, wsems):
    wid = lax.axis_index("s") * NC + lax.axis_index("c")
    b0 = wid * BPW

    pltpu.sync_copy(xt_hbm.at[:, pl.ds(b0, BPW)], idxt_v)

    lanes = lax.iota(jnp.int32, L)
    l_vecs = [lanes + (k * L) for k in range(BPW // L)]

    def fire_gather(h, b):
        pltpu.async_copy(
            table_hbm.at[idxt_v.at[h]], rows_v.at[b], gsems.at[b]
        )

    def drain_gather(b):
        pltpu.make_async_copy(
            table_hbm.at[pl.ds(0, BPW)], rows_v.at[b], gsems.at[b]
        ).wait()

    def transpose(b):
        for d in range(EMBED_DIM):
            d_vec = jnp.full((L,), d, jnp.int32)
            for k in range(BPW // L):
                vals = plsc.load_gather(rows_v.at[b], [l_vecs[k], d_vec])
                tr_v[b, d // 8, d % 8, pl.ds(k * L, L)] = vals

    def fire_write(h, b):
        pltpu.async_copy(
            tr_v.at[b], out_hbm.at[h].at[:, wid], wsems.at[b]
        )

    def drain_write(b):
        pltpu.make_async_copy(
            tr_v.at[b], out_hbm.at[0].at[:, 0], wsems.at[b]
        ).wait()

    for b in range(NBUF):
        fire_gather(b, b)

    @pl.loop(0, HIST, step=NBUF)
    def _steps(g):
        for b in range(NBUF):
            h = g + b
            drain_gather(b)

            @pl.when(h >= NBUF)
            def _():
                drain_write(b)

            transpose(b)
            fire_write(h, b)

            @pl.when(h + NBUF < HIST)
            def _():
                fire_gather(h + NBUF, b)

    for b in range(NBUF):
        drain_write(b)


@jax.jit
def _emb_call(xt, table):
    mesh = plsc.VectorSubcoreMesh(core_axis_name="c", subcore_axis_name="s")
    f = pl.kernel(
        _emb_body,
        out_type=jax.ShapeDtypeStruct((HIST, DT, NW, 8, BPW), jnp.float32),
        mesh=mesh,
        scratch_types=[
            pltpu.VMEM((HIST, BPW), jnp.int32),
            pltpu.VMEM((NBUF, BPW, EMBED_DIM), jnp.float32),
            pltpu.VMEM((NBUF, DT, 8, BPW), jnp.float32),
            pltpu.SemaphoreType.DMA((NBUF,)),
            pltpu.SemaphoreType.DMA((NBUF,)),
        ],
        compiler_params=pltpu.CompilerParams(
            use_tc_tiling_on_sc=False, needs_layout_passes=False
        ),
    )
    return f(xt, table)


def kernel(x, table):
    k = _emb_call(x.astype(jnp.int32).T, table)
    return k.transpose(2, 4, 0, 1, 3).reshape(BATCH, HIST, EMBED_DIM)


# speedup vs baseline: 1.4436x; 1.4436x over previous
import jax
import jax.numpy as jnp
from jax import lax
from jax.experimental import pallas as pl
from jax.experimental.pallas import tpu as pltpu
from jax.experimental.pallas import tpu_sc as plsc

VOCAB = 1000000
EMBED_DIM = 32
BATCH = 4096
HIST = 200

NC = 2
NS = 16
NW = NC * NS
BPW = BATCH // NW
DT = EMBED_DIM // 8
NBUF = 2
L = 16


def _emb_body(xt_hbm, table_hbm, out_hbm, idxt_v, rows_v, tr_v, gsems, wsems):
    wid = lax.axis_index("s") * NC + lax.axis_index("c")
    b0 = wid * BPW

    pltpu.sync_copy(xt_hbm.at[:, pl.ds(b0, BPW)], idxt_v)

    lanes = lax.iota(jnp.int32, L)
    l_vecs = [lanes + (k * L) for k in range(BPW // L)]

    def fire_gather(h, b):
        pltpu.async_copy(
            table_hbm.at[idxt_v.at[h]], rows_v.at[b], gsems.at[b]
        )

    def drain_gather(b):
        pltpu.make_async_copy(
            table_hbm.at[pl.ds(0, BPW)], rows_v.at[b], gsems.at[b]
        ).wait()

    def transpose(b):
        @plsc.parallel_loop(0, EMBED_DIM, step=1, unroll=8)
        def _d(d):
            d_vec = jnp.full((L,), 0, jnp.int32) + d
            dt = d // 8
            ds = lax.rem(d, 8)
            for k in range(BPW // L):
                vals = plsc.load_gather(rows_v.at[b], [l_vecs[k], d_vec])
                tr_v[b, dt, ds, pl.ds(k * L, L)] = vals

    def fire_write(h, b):
        pltpu.async_copy(
            tr_v.at[b], out_hbm.at[h].at[:, wid], wsems.at[b]
        )

    def drain_write(b):
        pltpu.make_async_copy(
            tr_v.at[b], out_hbm.at[0].at[:, 0], wsems.at[b]
        ).wait()

    for b in range(NBUF):
        fire_gather(b, b)

    @pl.loop(0, HIST, step=NBUF)
    def _steps(g):
        for b in range(NBUF):
            h = g + b
            drain_gather(b)

            @pl.when(h >= NBUF)
            def _():
                drain_write(b)

            transpose(b)
            fire_write(h, b)

            @pl.when(h + NBUF < HIST)
            def _():
                fire_gather(h + NBUF, b)

    for b in range(NBUF):
        drain_write(b)


@jax.jit
def _emb_call(xt, table):
    mesh = plsc.VectorSubcoreMesh(core_axis_name="c", subcore_axis_name="s")
    f = pl.kernel(
        _emb_body,
        out_type=jax.ShapeDtypeStruct((HIST, DT, NW, 8, BPW), jnp.float32),
        mesh=mesh,
        scratch_types=[
            pltpu.VMEM((HIST, BPW), jnp.int32),
            pltpu.VMEM((NBUF, BPW, EMBED_DIM), jnp.float32),
            pltpu.VMEM((NBUF, DT, 8, BPW), jnp.float32),
            pltpu.SemaphoreType.DMA((NBUF,)),
            pltpu.SemaphoreType.DMA((NBUF,)),
        ],
        compiler_params=pltpu.CompilerParams(
            use_tc_tiling_on_sc=False, needs_layout_passes=False
        ),
    )
    return f(xt, table)


def kernel(x, table):
    k = _emb_call(x.astype(jnp.int32).T, table)
    return k.transpose(2, 4, 0, 1, 3).reshape(BATCH, HIST, EMBED_DIM)
